# Initial kernel scaffold; baseline (speedup 1.0000x reference)
import jax
import jax.numpy as jnp
from jax.experimental import pallas as pl


def kernel(points, scale, freqs, cv):
    raise NotImplementedError("write your pallas kernel here")


# speedup vs baseline: 7.0413x; 7.0413x over previous
import dataclasses
import functools

import numpy as np
import jax
import jax.numpy as jnp
from jax import lax
from jax.experimental import pallas as pl
from jax.experimental.pallas import tpu as pltpu
from jax.experimental.pallas import tpu_sc as plsc

NUM_WORKERS = 32
P = 16
IDXW = 96


def _transpose_body(cv_ref, out_ref):
    out_ref[...] = cv_ref[0].T


def _meta_body(h, bands, pts_ref, m_ref, j0_ref, j1_ref, w0_ref, w1_ref, y_ref):
    pts = pts_ref[...]
    m = m_ref[...]
    fp = (pts[:, 0:1] * m[0:1, :] + pts[:, 1:2] * m[1:2, :]
          + pts[:, 2:3] * m[2:3, :])
    nb = fp.shape[0]
    col = lax.broadcasted_iota(jnp.int32, (nb, bands), 1)
    is_sin = ((col // 3) % 2) == 0
    y = jnp.where(is_sin, jnp.sin(fp), jnp.cos(fp))
    iy = (y + 1.0) * ((h - 1) * 0.5)
    i0f = jnp.floor(iy)
    fr = iy - i0f
    i1f = i0f + 1.0
    v0 = ((i0f >= 0.0) & (i0f <= h - 1.0)).astype(jnp.float32)
    v1 = ((i1f >= 0.0) & (i1f <= h - 1.0)).astype(jnp.float32)
    i0 = jnp.clip(i0f, 0.0, h - 1.0).astype(jnp.int32)
    i1 = jnp.clip(i1f, 0.0, h - 1.0).astype(jnp.int32)
    j0_ref[...] = i0 + col * h
    j1_ref[...] = i1 + col * h
    w0_ref[...] = (1.0 - fr) * v0
    w1_ref[...] = fr * v1
    y_ref[...] = y


def _sc_body(npw, nchunks, bands, c, cvt_hbm, j0_hbm, j1_hbm, meta_hbm,
             out_hbm, j0_v, j1_v, m_v, r0_v, r1_v, o_v, sem0, sem1):
    ncores = 2
    wid = lax.axis_index("s") * ncores + lax.axis_index("c")
    pt_base = wid * npw
    ngather = (P * bands) // IDXW

    @pl.loop(0, nchunks)
    def _chunk(ci):
        pt0 = pt_base + ci * P
        t0 = pt0 * bands
        pltpu.sync_copy(j0_hbm.at[pl.ds(t0, P * bands)], j0_v)
        pltpu.sync_copy(j1_hbm.at[pl.ds(t0, P * bands)], j1_v)
        pltpu.sync_copy(meta_hbm.at[pl.ds(t0 * 3, P * bands * 3)],
                        m_v.at[pl.ds(0, P * bands * 3)])
        waits = []
        for g in range(ngather):
            waits.append(pltpu.async_copy(
                cvt_hbm.at[j0_v.at[pl.ds(g * IDXW, IDXW)]],
                r0_v.at[pl.ds(g * IDXW, IDXW)], sem0))
            waits.append(pltpu.async_copy(
                cvt_hbm.at[j1_v.at[pl.ds(g * IDXW, IDXW)]],
                r1_v.at[pl.ds(g * IDXW, IDXW)], sem1))
        for w in waits:
            w.wait()

        lane = lax.iota(jnp.int32, 16)

        @pl.loop(0, P)
        def _pt(n):
            rowidx = jnp.full((16,), n, jnp.int32)

            @pl.loop(0, bands)
            def _band(b):
                t = n * bands + b
                tv = m_v[pl.ds(t * 3, 16)]
                w0s = tv[0]
                w1s = tv[1]
                ys = tv[2]
                for k in range(c // 16):
                    g0 = r0_v[t, pl.ds(k * 16, 16)]
                    g1 = r1_v[t, pl.ds(k * 16, 16)]
                    val = g0 * w0s + g1 * w1s + ys
                    colidx = lane * bands + (k * 16 * bands + b)
                    plsc.store_scatter(o_v, [rowidx, colidx], val)

        pltpu.sync_copy(o_v, out_hbm.at[pl.ds(pt0, P)])


def kernel(points, scale, freqs, cv):
    n = points.shape[0]
    f = freqs.shape[0]
    bands = f * 2 * 3
    c = cv.shape[1]
    h = cv.shape[2]
    cols = bands * c
    assert n % (NUM_WORKERS * P) == 0 and c % 16 == 0 and (P * bands) % IDXW == 0

    fidx = np.arange(bands) // (2 * 3)
    dsel = np.arange(bands) % 3
    onehot = jnp.asarray((dsel[None, :] == np.arange(3)[:, None]).astype(np.float32))
    m = onehot * (freqs[fidx][None, :] / scale)

    cvt = pl.pallas_call(
        _transpose_body,
        grid=(bands,),
        in_specs=[pl.BlockSpec((1, c, h), lambda b: (b, 0, 0))],
        out_specs=pl.BlockSpec((h, c), lambda b: (b, 0)),
        out_shape=jax.ShapeDtypeStruct((bands * h, c), jnp.float32),
    )(cv.reshape(bands, c, h))

    nb = 2048
    j0, j1, w0, w1, y = pl.pallas_call(
        functools.partial(_meta_body, h, bands),
        grid=(n // nb,),
        in_specs=[pl.BlockSpec((nb, 3), lambda i: (i, 0)),
                  pl.BlockSpec((3, bands), lambda i: (0, 0))],
        out_specs=[pl.BlockSpec((nb, bands), lambda i: (i, 0))] * 5,
        out_shape=[jax.ShapeDtypeStruct((n, bands), jnp.int32),
                   jax.ShapeDtypeStruct((n, bands), jnp.int32),
                   jax.ShapeDtypeStruct((n, bands), jnp.float32),
                   jax.ShapeDtypeStruct((n, bands), jnp.float32),
                   jax.ShapeDtypeStruct((n, bands), jnp.float32)],
    )(points, m)

    npw = n // NUM_WORKERS
    nchunks = npw // P
    ngather = (P * bands) // IDXW
    mesh = plsc.VectorSubcoreMesh(core_axis_name="c", subcore_axis_name="s")
    cp = pltpu.CompilerParams()
    if "needs_layout_passes" in pltpu.CompilerParams.__dataclass_fields__:
        cp = dataclasses.replace(cp, needs_layout_passes=False)
    if "use_tc_tiling_on_sc" in pltpu.CompilerParams.__dataclass_fields__:
        cp = dataclasses.replace(cp, use_tc_tiling_on_sc=False)
    sc = pl.kernel(
        functools.partial(_sc_body, npw, nchunks, bands, c),
        compiler_params=cp,
        out_type=jax.ShapeDtypeStruct((n, cols), jnp.float32),
        mesh=mesh,
        scratch_types=[
            pltpu.VMEM((P * bands,), jnp.int32),
            pltpu.VMEM((P * bands,), jnp.int32),
            pltpu.VMEM((P * bands * 3 + 16,), jnp.float32),
            pltpu.VMEM((P * bands, c), jnp.float32),
            pltpu.VMEM((P * bands, c), jnp.float32),
            pltpu.VMEM((P, cols), jnp.float32),
            pltpu.SemaphoreType.DMA,
            pltpu.SemaphoreType.DMA,
        ],
    )
    meta = jnp.stack([w0, w1, y], axis=-1).reshape(-1)
    return sc(cvt, j0.reshape(-1), j1.reshape(-1), meta)
